# Initial kernel scaffold; baseline (speedup 1.0000x reference)
import jax
import jax.numpy as jnp
from jax.experimental import pallas as pl


def kernel(x, edge_index, edge_attr, W1, b1, W2, b2):
    raise NotImplementedError("write your pallas kernel here")


# speedup vs baseline: 17.2201x; 17.2201x over previous
import functools

import jax
import jax.numpy as jnp
from jax import lax
from jax.experimental import pallas as pl
from jax.experimental.pallas import tpu as pltpu
from jax.experimental.pallas import tpu_sc as plsc

N = 10000
E = 320000
D_IN = 128
D_HID = 64
D_OUT = 128

NC = 2
NS = 16
NW = NC * NS
L = 16

N_PAD = 10240
ROWS_PER_TILE = N_PAD // NS
CHUNK = 128
E_TOT = E + N
CH = -(-E_TOT // (NW * CHUNK))
E_PAD = NW * CH * CHUNK
EPW = CH * CHUNK

_MESH = plsc.VectorSubcoreMesh(
    core_axis_name="c", subcore_axis_name="s", num_cores=NC, num_subcores=NS)


def _wid():
  return lax.axis_index("c") * NS + lax.axis_index("s")


def _zero_rows(buf, n_rows, d):
  z = jnp.zeros((L,), jnp.float32)
  def body(i, _):
    for k in range(d // L):
      buf[i, pl.ds(k * L, L)] = z
    return 0
  lax.fori_loop(0, n_rows, body, 0)


def _bcast16(x):
  return jnp.full((L,), x, dtype=jnp.int32)


def _deg_body(dst_hbm, ew_hbm, deg_out, dst_v, ew_v, vbuf, deg_sh):
  cid = lax.axis_index("c")
  sid = lax.axis_index("s")
  wid = _wid()
  base = sid * ROWS_PER_TILE

  z = jnp.zeros((L,), jnp.float32)
  def zb(i, _):
    vbuf[pl.ds(i * L, L)] = z
    return 0
  lax.fori_loop(0, ROWS_PER_TILE // L, zb, 0)
  pltpu.sync_copy(vbuf, deg_sh.at[pl.ds(base, ROWS_PER_TILE)])
  plsc.subcore_barrier()

  pltpu.sync_copy(dst_hbm.at[wid], dst_v)
  pltpu.sync_copy(ew_hbm.at[wid], ew_v)

  def chunk(j, _):
    pltpu.sync_copy(ew_v.at[j], deg_sh.at[dst_v.at[j]], add=True)
    return 0
  lax.fori_loop(0, CH, chunk, 0)
  plsc.subcore_barrier()

  pltpu.sync_copy(deg_sh.at[pl.ds(base, ROWS_PER_TILE)], vbuf)
  pltpu.sync_copy(vbuf, deg_out.at[cid, pl.ds(base, ROWS_PER_TILE)])


_deg_kernel = pl.kernel(
    _deg_body,
    out_type=jax.ShapeDtypeStruct((NC, N_PAD), jnp.float32),
    mesh=_MESH,
    compiler_params=pltpu.CompilerParams(needs_layout_passes=False, use_tc_tiling_on_sc=False),
    scratch_types=[
        pltpu.VMEM((CH, CHUNK), jnp.int32),
        pltpu.VMEM((CH, CHUNK), jnp.float32),
        pltpu.VMEM((ROWS_PER_TILE,), jnp.float32),
        pltpu.VMEM_SHARED((N_PAD,), jnp.float32),
    ],
)


def _rsqrt16(x):
  i = lax.bitcast_convert_type(x, jnp.int32)
  i = jnp.int32(0x5F3759DF) - lax.shift_right_logical(i, 1)
  y = lax.bitcast_convert_type(i, jnp.float32)
  for _ in range(4):
    y = y * (1.5 - 0.5 * x * y * y)
  return y


def _norm_body(deg_hbm, src_hbm, dst_hbm, ew_hbm, norm_out,
               src_v, dst_v, ew_v, norm_v, d0_v, d1_v, dis_v, dis_sh):
  cid = lax.axis_index("c")
  sid = lax.axis_index("s")
  wid = _wid()
  base = sid * ROWS_PER_TILE

  pltpu.sync_copy(deg_hbm.at[0, pl.ds(base, ROWS_PER_TILE)], d0_v)
  pltpu.sync_copy(deg_hbm.at[1, pl.ds(base, ROWS_PER_TILE)], d1_v)

  def disb(i, _):
    sl = pl.ds(i * L, L)
    d = d0_v[sl] + d1_v[sl]
    d = jnp.maximum(d, 1e-12)
    d0_v[sl] = _rsqrt16(d)
    return 0
  lax.fori_loop(0, ROWS_PER_TILE // L, disb, 0)
  pltpu.sync_copy(d0_v, dis_sh.at[pl.ds(base, ROWS_PER_TILE)])
  plsc.subcore_barrier()

  pltpu.sync_copy(dis_sh, dis_v)
  pltpu.sync_copy(src_hbm.at[wid], src_v)
  pltpu.sync_copy(dst_hbm.at[wid], dst_v)
  pltpu.sync_copy(ew_hbm.at[wid], ew_v)

  def chunk(j, _):
    def grp(g, _):
      sl = pl.ds(g * L, L)
      s16 = src_v[j, sl]
      d16 = dst_v[j, sl]
      ds_ = plsc.load_gather(dis_v, [s16])
      dd_ = plsc.load_gather(dis_v, [d16])
      norm_v[j, sl] = ds_ * ew_v[j, sl] * dd_
      return 0
    lax.fori_loop(0, CHUNK // L, grp, 0)
    return 0
  lax.fori_loop(0, CH, chunk, 0)

  pltpu.sync_copy(norm_v, norm_out.at[wid])


_norm_kernel = pl.kernel(
    _norm_body,
    out_type=jax.ShapeDtypeStruct((NW, CH, CHUNK), jnp.float32),
    mesh=_MESH,
    compiler_params=pltpu.CompilerParams(needs_layout_passes=False, use_tc_tiling_on_sc=False),
    scratch_types=[
        pltpu.VMEM((CH, CHUNK), jnp.int32),
        pltpu.VMEM((CH, CHUNK), jnp.int32),
        pltpu.VMEM((CH, CHUNK), jnp.float32),
        pltpu.VMEM((CH, CHUNK), jnp.float32),
        pltpu.VMEM((ROWS_PER_TILE,), jnp.float32),
        pltpu.VMEM((ROWS_PER_TILE,), jnp.float32),
        pltpu.VMEM((N_PAD,), jnp.float32),
        pltpu.VMEM_SHARED((N_PAD,), jnp.float32),
    ],
)


def _msg_body(d, h_hbm, src_hbm, dst_hbm, norm_hbm, acc_out,
              src_v, dst_v, norm_v, rows_v, acc_sh, sem):
  cid = lax.axis_index("c")
  sid = lax.axis_index("s")
  wid = _wid()
  base = sid * ROWS_PER_TILE

  _zero_rows(rows_v, CHUNK, d)
  for k in range(ROWS_PER_TILE // CHUNK):
    pltpu.sync_copy(rows_v, acc_sh.at[pl.ds(base + k * CHUNK, CHUNK), :])
  plsc.subcore_barrier()

  pltpu.sync_copy(src_hbm.at[wid], src_v)
  pltpu.sync_copy(dst_hbm.at[wid], dst_v)
  pltpu.sync_copy(norm_hbm.at[wid], norm_v)

  def chunk(j, _):
    pltpu.async_copy(h_hbm.at[src_v.at[j]], rows_v, sem).wait()
    def scale(e, _):
      nb = plsc.load_gather(norm_v, [_bcast16(j), _bcast16(e)])
      for k in range(d // L):
        sl = pl.ds(k * L, L)
        rows_v[e, sl] = rows_v[e, sl] * nb
      return 0
    lax.fori_loop(0, CHUNK, scale, 0)
    pltpu.sync_copy(rows_v, acc_sh.at[dst_v.at[j]], add=True)
    return 0
  lax.fori_loop(0, CH, chunk, 0)
  plsc.subcore_barrier()

  for k in range(ROWS_PER_TILE // CHUNK):
    sl = pl.ds(base + k * CHUNK, CHUNK)
    pltpu.sync_copy(acc_sh.at[sl, :], rows_v)
    pltpu.sync_copy(rows_v, acc_out.at[cid, sl, :])


def _make_msg_kernel(d):
  return pl.kernel(
      functools.partial(_msg_body, d),
      out_type=jax.ShapeDtypeStruct((NC, N_PAD, d), jnp.float32),
      mesh=_MESH,
      compiler_params=pltpu.CompilerParams(needs_layout_passes=False, use_tc_tiling_on_sc=False),
      scratch_types=[
          pltpu.VMEM((CH, CHUNK), jnp.int32),
          pltpu.VMEM((CH, CHUNK), jnp.int32),
          pltpu.VMEM((CH, CHUNK), jnp.float32),
          pltpu.VMEM((CHUNK, d), jnp.float32),
          pltpu.VMEM_SHARED((N_PAD, d), jnp.float32),
          pltpu.SemaphoreType.DMA,
      ],
  )


_msg_hid = _make_msg_kernel(D_HID)
_msg_out = _make_msg_kernel(D_OUT)


ROWS_BLK = 2000


def _mm1_body(x_ref, w_ref, o_ref):
  o_ref[...] = jnp.dot(x_ref[...], w_ref[...],
                       preferred_element_type=jnp.float32)


def _tc_matmul1(x, w1):
  return pl.pallas_call(
      _mm1_body,
      grid=(N // ROWS_BLK,),
      in_specs=[
          pl.BlockSpec((ROWS_BLK, D_IN), lambda i: (i, 0)),
          pl.BlockSpec((D_IN, D_HID), lambda i: (0, 0)),
      ],
      out_specs=pl.BlockSpec((ROWS_BLK, D_HID), lambda i: (i, 0)),
      out_shape=jax.ShapeDtypeStruct((N, D_HID), jnp.float32),
  )(x, w1)


def _mid_body(acc_ref, b1_ref, w2_ref, o_ref):
  z = acc_ref[0] + acc_ref[1] + b1_ref[...]
  alpha = 1.6732632423543772
  scale = 1.0507009873554805
  z = scale * jnp.where(z > 0, z, alpha * (jnp.exp(z) - 1.0))
  o_ref[...] = jnp.dot(z, w2_ref[...], preferred_element_type=jnp.float32)


def _tc_mid(acc1, b1, w2):
  return pl.pallas_call(
      _mid_body,
      grid=(N // ROWS_BLK,),
      in_specs=[
          pl.BlockSpec((NC, ROWS_BLK, D_HID), lambda i: (0, i, 0)),
          pl.BlockSpec((1, D_HID), lambda i: (0, 0)),
          pl.BlockSpec((D_HID, D_OUT), lambda i: (0, 0)),
      ],
      out_specs=pl.BlockSpec((ROWS_BLK, D_OUT), lambda i: (i, 0)),
      out_shape=jax.ShapeDtypeStruct((N, D_OUT), jnp.float32),
  )(acc1, b1, w2)


def _fin_body(acc_ref, b2_ref, o_ref):
  z = acc_ref[0] + acc_ref[1] + b2_ref[...]
  m = jnp.max(z, axis=-1, keepdims=True)
  ez = jnp.exp(z - m)
  o_ref[...] = ez / jnp.sum(ez, axis=-1, keepdims=True)


def _tc_fin(acc2, b2):
  return pl.pallas_call(
      _fin_body,
      grid=(N // ROWS_BLK,),
      in_specs=[
          pl.BlockSpec((NC, ROWS_BLK, D_OUT), lambda i: (0, i, 0)),
          pl.BlockSpec((1, D_OUT), lambda i: (0, 0)),
      ],
      out_specs=pl.BlockSpec((ROWS_BLK, D_OUT), lambda i: (i, 0)),
      out_shape=jax.ShapeDtypeStruct((N, D_OUT), jnp.float32),
  )(acc2, b2)


def kernel(x, edge_index, edge_attr, W1, b1, W2, b2):
  pad = E_PAD - E_TOT
  loop = jnp.arange(N, dtype=jnp.int32)
  pad_src = jnp.arange(pad, dtype=jnp.int32) % N
  pad_dst = N + jnp.arange(pad, dtype=jnp.int32) % (N_PAD - N)
  src = jnp.concatenate([edge_index[0], loop, pad_src]).reshape(NW, CH, CHUNK)
  dst = jnp.concatenate([edge_index[1], loop, pad_dst]).reshape(NW, CH, CHUNK)
  ew = jnp.concatenate(
      [edge_attr, jnp.ones((N,), jnp.float32), jnp.zeros((pad,), jnp.float32)]
  ).reshape(NW, CH, CHUNK)

  deg_p = _deg_kernel(dst, ew)
  norm = _norm_kernel(deg_p, src, dst, ew)

  h1 = _tc_matmul1(x, W1)
  acc1 = _msg_hid(h1, src, dst, norm)
  h2 = _tc_mid(acc1, b1.reshape(1, D_HID), W2)
  acc2 = _msg_out(h2, src, dst, norm)
  out = _tc_fin(acc2, b2.reshape(1, D_OUT))
  return out


# speedup vs baseline: 20.6792x; 1.2009x over previous
import functools

import jax
import jax.numpy as jnp
from jax import lax
from jax.experimental import pallas as pl
from jax.experimental.pallas import tpu as pltpu
from jax.experimental.pallas import tpu_sc as plsc

N = 10000
E = 320000
D_IN = 128
D_HID = 64
D_OUT = 128

NC = 2
NS = 16
NW = NC * NS
L = 16

N_PAD = 10240
ROWS_PER_TILE = N_PAD // NS
H_ROWS_PER_TILE = N // NS
CHUNK = 128
E_TOT = E + N
CH = 81
E_PAD = NW * CH * CHUNK
GRPS = CH // 3

_MESH = plsc.VectorSubcoreMesh(
    core_axis_name="c", subcore_axis_name="s", num_cores=NC, num_subcores=NS)
_SC_PARAMS = pltpu.CompilerParams(
    needs_layout_passes=False, use_tc_tiling_on_sc=False)

_BCAST_DN = lax.GatherDimensionNumbers(
    offset_dims=(), collapsed_slice_dims=(0,), start_index_map=(0,))


def _lane_bcast(v16, lane):
  idx = jnp.full((L, 1), lane, jnp.int32)
  return lax.gather(v16, idx, _BCAST_DN, slice_sizes=(1,),
                    mode=lax.GatherScatterMode.PROMISE_IN_BOUNDS)


def _wid():
  return lax.axis_index("c") * NS + lax.axis_index("s")


def _deg_body(dst_hbm, ew_hbm, deg_out, dst_v, ew_v, vbuf, deg_sh):
  cid = lax.axis_index("c")
  sid = lax.axis_index("s")
  wid = _wid()
  base = sid * ROWS_PER_TILE

  z = jnp.zeros((L,), jnp.float32)
  def zb(i, _):
    vbuf[pl.ds(i * L, L)] = z
    return 0
  lax.fori_loop(0, ROWS_PER_TILE // L, zb, 0)
  pltpu.sync_copy(vbuf, deg_sh.at[pl.ds(base, ROWS_PER_TILE)])
  plsc.subcore_barrier()

  pltpu.sync_copy(dst_hbm.at[wid], dst_v)
  pltpu.sync_copy(ew_hbm.at[wid], ew_v)

  def chunk(j, _):
    pltpu.sync_copy(ew_v.at[j], deg_sh.at[dst_v.at[j]], add=True)
    return 0
  lax.fori_loop(0, CH, chunk, 0)
  plsc.subcore_barrier()

  pltpu.sync_copy(deg_sh.at[pl.ds(base, ROWS_PER_TILE)], vbuf)
  pltpu.sync_copy(vbuf, deg_out.at[cid, pl.ds(base, ROWS_PER_TILE)])


_deg_kernel = pl.kernel(
    _deg_body,
    out_type=jax.ShapeDtypeStruct((NC, N_PAD), jnp.float32),
    mesh=_MESH,
    compiler_params=_SC_PARAMS,
    scratch_types=[
        pltpu.VMEM((CH, CHUNK), jnp.int32),
        pltpu.VMEM((CH, CHUNK), jnp.float32),
        pltpu.VMEM((ROWS_PER_TILE,), jnp.float32),
        pltpu.VMEM_SHARED((N_PAD,), jnp.float32),
    ],
)


def _rsqrt16(x):
  i = lax.bitcast_convert_type(x, jnp.int32)
  i = jnp.int32(0x5F3759DF) - lax.shift_right_logical(i, 1)
  y = lax.bitcast_convert_type(i, jnp.float32)
  for _ in range(4):
    y = y * (1.5 - 0.5 * x * y * y)
  return y


def _norm_body(deg_hbm, src_hbm, dst_hbm, ew_hbm, norm_out,
               src_v, dst_v, ew_v, norm_v, d0_v, d1_v, dis_v, dis_sh):
  cid = lax.axis_index("c")
  sid = lax.axis_index("s")
  wid = _wid()
  base = sid * ROWS_PER_TILE

  pltpu.sync_copy(deg_hbm.at[0, pl.ds(base, ROWS_PER_TILE)], d0_v)
  pltpu.sync_copy(deg_hbm.at[1, pl.ds(base, ROWS_PER_TILE)], d1_v)

  def disb(i, _):
    sl = pl.ds(i * L, L)
    d = d0_v[sl] + d1_v[sl]
    d = jnp.maximum(d, 1e-12)
    d0_v[sl] = _rsqrt16(d)
    return 0
  lax.fori_loop(0, ROWS_PER_TILE // L, disb, 0)
  pltpu.sync_copy(d0_v, dis_sh.at[pl.ds(base, ROWS_PER_TILE)])
  plsc.subcore_barrier()

  pltpu.sync_copy(dis_sh, dis_v)
  pltpu.sync_copy(src_hbm.at[wid], src_v)
  pltpu.sync_copy(dst_hbm.at[wid], dst_v)
  pltpu.sync_copy(ew_hbm.at[wid], ew_v)

  def chunk(j, _):
    def grp(g, _):
      sl = pl.ds(g * L, L)
      s16 = src_v[j, sl]
      d16 = dst_v[j, sl]
      ds_ = plsc.load_gather(dis_v, [s16])
      dd_ = plsc.load_gather(dis_v, [d16])
      norm_v[j, sl] = ds_ * ew_v[j, sl] * dd_
      return 0
    lax.fori_loop(0, CHUNK // L, grp, 0)
    return 0
  lax.fori_loop(0, CH, chunk, 0)

  pltpu.sync_copy(norm_v, norm_out.at[wid])


_norm_kernel = pl.kernel(
    _norm_body,
    out_type=jax.ShapeDtypeStruct((NW, CH, CHUNK), jnp.float32),
    mesh=_MESH,
    compiler_params=_SC_PARAMS,
    scratch_types=[
        pltpu.VMEM((CH, CHUNK), jnp.int32),
        pltpu.VMEM((CH, CHUNK), jnp.int32),
        pltpu.VMEM((CH, CHUNK), jnp.float32),
        pltpu.VMEM((CH, CHUNK), jnp.float32),
        pltpu.VMEM((ROWS_PER_TILE,), jnp.float32),
        pltpu.VMEM((ROWS_PER_TILE,), jnp.float32),
        pltpu.VMEM((N_PAD,), jnp.float32),
        pltpu.VMEM_SHARED((N_PAD,), jnp.float32),
    ],
)


def _msg_body(h_hbm, src_hbm, dst_hbm, norm_hbm, acc_out,
              src_v, dst_v, norm_v, rows0, rows1, rows2,
              acc_sh, gsem0, gsem1, gsem2, ssem0, ssem1, ssem2):
  cid = lax.axis_index("c")
  sid = lax.axis_index("s")
  wid = _wid()
  base = sid * ROWS_PER_TILE

  bufs = (rows0, rows1, rows2)
  gsems = (gsem0, gsem1, gsem2)
  ssems = (ssem0, ssem1, ssem2)

  z = jnp.zeros((L,), jnp.float32)
  def zb(i, _):
    for k in range(D_HID // L):
      rows0[i, pl.ds(k * L, L)] = z
    return 0
  lax.fori_loop(0, CHUNK, zb, 0)
  for k in range(ROWS_PER_TILE // CHUNK):
    pltpu.sync_copy(rows0, acc_sh.at[pl.ds(base + k * CHUNK, CHUNK), :])
  plsc.subcore_barrier()

  pltpu.sync_copy(src_hbm.at[wid], src_v)
  pltpu.sync_copy(dst_hbm.at[wid], dst_v)
  pltpu.sync_copy(norm_hbm.at[wid], norm_v)

  def scale(buf, j):
    def grp(g16, _):
      n16 = norm_v[j, pl.ds(g16 * L, L)]
      for e16 in range(L):
        nb = _lane_bcast(n16, e16)
        e = g16 * L + e16
        for k in range(D_HID // L):
          sl = pl.ds(k * L, L)
          buf[e, sl] = buf[e, sl] * nb
      return 0
    lax.fori_loop(0, CHUNK // L, grp, 0)

  def gather(j, b):
    pltpu.async_copy(h_hbm.at[src_v.at[j]], bufs[b], gsems[b])

  def wait_gather(j, b):
    pltpu.make_async_copy(h_hbm.at[src_v.at[j]], bufs[b], gsems[b]).wait()

  def scatter(j, b):
    pltpu.async_copy(bufs[b], acc_sh.at[dst_v.at[j]], ssems[b], add=True)

  def wait_scatter(j, b):
    pltpu.make_async_copy(bufs[b], acc_sh.at[dst_v.at[j]], ssems[b]).wait()

  def step(j, b, issue_gather, wait_prev_scatter):
    wait_gather(j, b)
    scale(bufs[b], j)
    scatter(j, b)
    if issue_gather:
      b2 = (b + 2) % 3
      if wait_prev_scatter:
        wait_scatter(j - 1, b2)
      gather(j + 2, b2)

  gather(jnp.int32(0), 0)
  gather(jnp.int32(1), 1)

  step(jnp.int32(0), 0, True, False)
  step(jnp.int32(1), 1, True, True)
  step(jnp.int32(2), 2, True, True)

  def group(g, _):
    j0 = g * 3
    step(j0, 0, True, True)
    step(j0 + 1, 1, True, True)
    step(j0 + 2, 2, True, True)
    return 0
  lax.fori_loop(1, GRPS - 1, group, 0)

  j0 = jnp.int32(CH - 3)
  step(j0, 0, True, True)
  step(j0 + 1, 1, False, False)
  step(j0 + 2, 2, False, False)
  for b in range(3):
    wait_scatter(jnp.int32(CH - 3 + b), b)

  plsc.subcore_barrier()

  for k in range(ROWS_PER_TILE // CHUNK):
    sl = pl.ds(base + k * CHUNK, CHUNK)
    pltpu.sync_copy(acc_sh.at[sl, :], rows0)
    pltpu.sync_copy(rows0, acc_out.at[cid, sl, :])


_msg_kernel = pl.kernel(
    _msg_body,
    out_type=jax.ShapeDtypeStruct((NC, N_PAD, D_HID), jnp.float32),
    mesh=_MESH,
    compiler_params=_SC_PARAMS,
    scratch_types=[
        pltpu.VMEM((CH, CHUNK), jnp.int32),
        pltpu.VMEM((CH, CHUNK), jnp.int32),
        pltpu.VMEM((CH, CHUNK), jnp.float32),
        pltpu.VMEM((CHUNK, D_HID), jnp.float32),
        pltpu.VMEM((CHUNK, D_HID), jnp.float32),
        pltpu.VMEM((CHUNK, D_HID), jnp.float32),
        pltpu.VMEM_SHARED((N_PAD, D_HID), jnp.float32),
        pltpu.SemaphoreType.DMA,
        pltpu.SemaphoreType.DMA,
        pltpu.SemaphoreType.DMA,
        pltpu.SemaphoreType.DMA,
        pltpu.SemaphoreType.DMA,
        pltpu.SemaphoreType.DMA,
    ],
)


ROWS_BLK = 2000


def _mm1_body(x_ref, w_ref, o_ref):
  o_ref[...] = jnp.dot(x_ref[...], w_ref[...],
                       preferred_element_type=jnp.float32)


def _tc_matmul1(x, w1):
  return pl.pallas_call(
      _mm1_body,
      grid=(N // ROWS_BLK,),
      in_specs=[
          pl.BlockSpec((ROWS_BLK, D_IN), lambda i: (i, 0)),
          pl.BlockSpec((D_IN, D_HID), lambda i: (0, 0)),
      ],
      out_specs=pl.BlockSpec((ROWS_BLK, D_HID), lambda i: (i, 0)),
      out_shape=jax.ShapeDtypeStruct((N, D_HID), jnp.float32),
  )(x, w1)


def _selu_body(acc_ref, b1_ref, o_ref):
  z = acc_ref[0] + acc_ref[1] + b1_ref[...]
  alpha = 1.6732632423543772
  scale = 1.0507009873554805
  o_ref[...] = scale * jnp.where(z > 0, z, alpha * (jnp.exp(z) - 1.0))


def _tc_selu(acc1, b1):
  return pl.pallas_call(
      _selu_body,
      grid=(N // ROWS_BLK,),
      in_specs=[
          pl.BlockSpec((NC, ROWS_BLK, D_HID), lambda i: (0, i, 0)),
          pl.BlockSpec((1, D_HID), lambda i: (0, 0)),
      ],
      out_specs=pl.BlockSpec((ROWS_BLK, D_HID), lambda i: (i, 0)),
      out_shape=jax.ShapeDtypeStruct((N, D_HID), jnp.float32),
  )(acc1, b1)


def _fin_body(acc_ref, w2_ref, b2_ref, o_ref):
  zin = acc_ref[0] + acc_ref[1]
  z = jnp.dot(zin, w2_ref[...], preferred_element_type=jnp.float32)
  z = z + b2_ref[...]
  m = jnp.max(z, axis=-1, keepdims=True)
  ez = jnp.exp(z - m)
  o_ref[...] = ez / jnp.sum(ez, axis=-1, keepdims=True)


def _tc_fin(acc2, w2, b2):
  return pl.pallas_call(
      _fin_body,
      grid=(N // ROWS_BLK,),
      in_specs=[
          pl.BlockSpec((NC, ROWS_BLK, D_HID), lambda i: (0, i, 0)),
          pl.BlockSpec((D_HID, D_OUT), lambda i: (0, 0)),
          pl.BlockSpec((1, D_OUT), lambda i: (0, 0)),
      ],
      out_specs=pl.BlockSpec((ROWS_BLK, D_OUT), lambda i: (i, 0)),
      out_shape=jax.ShapeDtypeStruct((N, D_OUT), jnp.float32),
  )(acc2, w2, b2)


def kernel(x, edge_index, edge_attr, W1, b1, W2, b2):
  pad = E_PAD - E_TOT
  loop = jnp.arange(N, dtype=jnp.int32)
  pad_src = jnp.arange(pad, dtype=jnp.int32) % N
  pad_dst = N + jnp.arange(pad, dtype=jnp.int32) % (N_PAD - N)
  src = jnp.concatenate([edge_index[0], loop, pad_src]).reshape(NW, CH, CHUNK)
  dst = jnp.concatenate([edge_index[1], loop, pad_dst]).reshape(NW, CH, CHUNK)
  ew = jnp.concatenate(
      [edge_attr, jnp.ones((N,), jnp.float32), jnp.zeros((pad,), jnp.float32)]
  ).reshape(NW, CH, CHUNK)

  deg_p = _deg_kernel(dst, ew)
  norm = _norm_kernel(deg_p, src, dst, ew)

  h1 = _tc_matmul1(x, W1)
  acc1 = _msg_kernel(h1, src, dst, norm)
  z = _tc_selu(acc1, b1.reshape(1, D_HID))
  acc2 = _msg_kernel(z, src, dst, norm)
  out = _tc_fin(acc2, W2, b2.reshape(1, D_OUT))
  return out


# speedup vs baseline: 36.6663x; 1.7731x over previous
import functools

import jax
import jax.numpy as jnp
from jax import lax
from jax.experimental import pallas as pl
from jax.experimental.pallas import tpu as pltpu
from jax.experimental.pallas import tpu_sc as plsc

N = 10000
E = 320000
D_IN = 128
D_HID = 64
D_OUT = 128

NC = 2
NS = 16
NW = NC * NS
L = 16

N_PAD = 10240
ROWS_PER_TILE = N_PAD // NS
H_ROWS_PER_TILE = N // NS
CHUNK = 128
E_TOT = E + N
CH = 81
E_PAD = NW * CH * CHUNK
GRPS = CH // 3

_MESH = plsc.VectorSubcoreMesh(
    core_axis_name="c", subcore_axis_name="s", num_cores=NC, num_subcores=NS)
_SC_PARAMS = pltpu.CompilerParams(
    needs_layout_passes=False, use_tc_tiling_on_sc=False)

_BCAST_DN = lax.GatherDimensionNumbers(
    offset_dims=(), collapsed_slice_dims=(0,), start_index_map=(0,))


def _lane_bcast(v16, lane):
  idx = jnp.full((L, 1), lane, jnp.int32)
  return lax.gather(v16, idx, _BCAST_DN, slice_sizes=(1,),
                    mode=lax.GatherScatterMode.PROMISE_IN_BOUNDS)


def _wid():
  return lax.axis_index("c") * NS + lax.axis_index("s")


def _deg_body(dst_hbm, ew_hbm, deg_out, dst_v, ew_v, vbuf, deg_sh):
  cid = lax.axis_index("c")
  sid = lax.axis_index("s")
  wid = _wid()
  base = sid * ROWS_PER_TILE

  z = jnp.zeros((L,), jnp.float32)
  def zb(i, _):
    vbuf[pl.ds(i * L, L)] = z
    return 0
  lax.fori_loop(0, ROWS_PER_TILE // L, zb, 0)
  pltpu.sync_copy(vbuf, deg_sh.at[pl.ds(base, ROWS_PER_TILE)])
  plsc.subcore_barrier()

  pltpu.sync_copy(dst_hbm.at[wid], dst_v)
  pltpu.sync_copy(ew_hbm.at[wid], ew_v)

  def chunk(j, _):
    pltpu.sync_copy(ew_v.at[j], deg_sh.at[dst_v.at[j]], add=True)
    return 0
  lax.fori_loop(0, CH, chunk, 0)
  plsc.subcore_barrier()

  pltpu.sync_copy(deg_sh.at[pl.ds(base, ROWS_PER_TILE)], vbuf)
  pltpu.sync_copy(vbuf, deg_out.at[cid, pl.ds(base, ROWS_PER_TILE)])


_deg_kernel = pl.kernel(
    _deg_body,
    out_type=jax.ShapeDtypeStruct((NC, N_PAD), jnp.float32),
    mesh=_MESH,
    compiler_params=_SC_PARAMS,
    scratch_types=[
        pltpu.VMEM((CH, CHUNK), jnp.int32),
        pltpu.VMEM((CH, CHUNK), jnp.float32),
        pltpu.VMEM((ROWS_PER_TILE,), jnp.float32),
        pltpu.VMEM_SHARED((N_PAD,), jnp.float32),
    ],
)


def _rsqrt16(x):
  i = lax.bitcast_convert_type(x, jnp.int32)
  i = jnp.int32(0x5F3759DF) - lax.shift_right_logical(i, 1)
  y = lax.bitcast_convert_type(i, jnp.float32)
  for _ in range(4):
    y = y * (1.5 - 0.5 * x * y * y)
  return y


def _norm_body(deg_hbm, src_hbm, dst_hbm, ew_hbm, norm_out,
               src_v, dst_v, ew_v, norm_v, d0_v, d1_v, dis_v, dis_sh):
  cid = lax.axis_index("c")
  sid = lax.axis_index("s")
  wid = _wid()
  base = sid * ROWS_PER_TILE

  pltpu.sync_copy(deg_hbm.at[0, pl.ds(base, ROWS_PER_TILE)], d0_v)
  pltpu.sync_copy(deg_hbm.at[1, pl.ds(base, ROWS_PER_TILE)], d1_v)

  def disb(i, _):
    sl = pl.ds(i * L, L)
    d = d0_v[sl] + d1_v[sl]
    d = jnp.maximum(d, 1e-12)
    d0_v[sl] = _rsqrt16(d)
    return 0
  lax.fori_loop(0, ROWS_PER_TILE // L, disb, 0)
  pltpu.sync_copy(d0_v, dis_sh.at[pl.ds(base, ROWS_PER_TILE)])
  plsc.subcore_barrier()

  pltpu.sync_copy(dis_sh, dis_v)
  pltpu.sync_copy(src_hbm.at[wid], src_v)
  pltpu.sync_copy(dst_hbm.at[wid], dst_v)
  pltpu.sync_copy(ew_hbm.at[wid], ew_v)

  def chunk(j, _):
    def grp(g, _):
      sl = pl.ds(g * L, L)
      s16 = src_v[j, sl]
      d16 = dst_v[j, sl]
      ds_ = plsc.load_gather(dis_v, [s16])
      dd_ = plsc.load_gather(dis_v, [d16])
      norm_v[j, sl] = ds_ * ew_v[j, sl] * dd_
      return 0
    lax.fori_loop(0, CHUNK // L, grp, 0)
    return 0
  lax.fori_loop(0, CH, chunk, 0)

  pltpu.sync_copy(norm_v, norm_out.at[wid])


_norm_kernel = pl.kernel(
    _norm_body,
    out_type=jax.ShapeDtypeStruct((NW, CH, CHUNK), jnp.float32),
    mesh=_MESH,
    compiler_params=_SC_PARAMS,
    scratch_types=[
        pltpu.VMEM((CH, CHUNK), jnp.int32),
        pltpu.VMEM((CH, CHUNK), jnp.int32),
        pltpu.VMEM((CH, CHUNK), jnp.float32),
        pltpu.VMEM((CH, CHUNK), jnp.float32),
        pltpu.VMEM((ROWS_PER_TILE,), jnp.float32),
        pltpu.VMEM((ROWS_PER_TILE,), jnp.float32),
        pltpu.VMEM((N_PAD,), jnp.float32),
        pltpu.VMEM_SHARED((N_PAD,), jnp.float32),
    ],
)


def _msg_body(h_hbm, src_hbm, dst_hbm, norm_hbm, acc_out,
              src_v, dst_v, norm_v, rows0, rows1, rows2,
              acc_sh, gsem0, gsem1, gsem2, ssem0, ssem1, ssem2):
  cid = lax.axis_index("c")
  sid = lax.axis_index("s")
  wid = _wid()
  base = sid * ROWS_PER_TILE

  bufs = (rows0, rows1, rows2)
  gsems = (gsem0, gsem1, gsem2)
  ssems = (ssem0, ssem1, ssem2)

  z = jnp.zeros((L,), jnp.float32)
  def zb(i, _):
    for k in range(D_HID // L):
      rows0[i, pl.ds(k * L, L)] = z
    return 0
  lax.fori_loop(0, CHUNK, zb, 0)
  for k in range(ROWS_PER_TILE // CHUNK):
    pltpu.sync_copy(rows0, acc_sh.at[pl.ds(base + k * CHUNK, CHUNK), :])
  plsc.subcore_barrier()

  pltpu.sync_copy(src_hbm.at[wid], src_v)
  pltpu.sync_copy(dst_hbm.at[wid], dst_v)
  pltpu.sync_copy(norm_hbm.at[wid], norm_v)

  def scale(buf, j):
    nk = D_HID // L
    def grp(g16, _):
      n16 = norm_v[j, pl.ds(g16 * L, L)]
      for e16 in range(0, L, 2):
        ea = g16 * L + e16
        eb = ea + 1
        nba = _lane_bcast(n16, e16)
        nbb = _lane_bcast(n16, e16 + 1)
        va = [buf[ea, pl.ds(k * L, L)] for k in range(nk)]
        vb = [buf[eb, pl.ds(k * L, L)] for k in range(nk)]
        for k in range(nk):
          buf[ea, pl.ds(k * L, L)] = va[k] * nba
        for k in range(nk):
          buf[eb, pl.ds(k * L, L)] = vb[k] * nbb
      return 0
    lax.fori_loop(0, CHUNK // L, grp, 0)

  def gather(j, b):
    pltpu.async_copy(h_hbm.at[src_v.at[j]], bufs[b], gsems[b])

  def wait_gather(j, b):
    pltpu.make_async_copy(h_hbm.at[src_v.at[j]], bufs[b], gsems[b]).wait()

  def scatter(j, b):
    pltpu.async_copy(bufs[b], acc_sh.at[dst_v.at[j]], ssems[b], add=True)

  def wait_scatter(j, b):
    pltpu.make_async_copy(bufs[b], acc_sh.at[dst_v.at[j]], ssems[b]).wait()

  def step(j, b, issue_gather, wait_prev_scatter):
    wait_gather(j, b)
    scale(bufs[b], j)
    scatter(j, b)
    if issue_gather:
      b2 = (b + 2) % 3
      if wait_prev_scatter:
        wait_scatter(j - 1, b2)
      gather(j + 2, b2)

  gather(jnp.int32(0), 0)
  gather(jnp.int32(1), 1)

  step(jnp.int32(0), 0, True, False)
  step(jnp.int32(1), 1, True, True)
  step(jnp.int32(2), 2, True, True)

  def group(g, _):
    j0 = g * 3
    step(j0, 0, True, True)
    step(j0 + 1, 1, True, True)
    step(j0 + 2, 2, True, True)
    return 0
  lax.fori_loop(1, GRPS - 1, group, 0)

  j0 = jnp.int32(CH - 3)
  step(j0, 0, True, True)
  step(j0 + 1, 1, False, False)
  step(j0 + 2, 2, False, False)
  for b in range(3):
    wait_scatter(jnp.int32(CH - 3 + b), b)

  plsc.subcore_barrier()

  for k in range(ROWS_PER_TILE // CHUNK):
    sl = pl.ds(base + k * CHUNK, CHUNK)
    pltpu.sync_copy(acc_sh.at[sl, :], rows0)
    pltpu.sync_copy(rows0, acc_out.at[cid, sl, :])


_msg_kernel = pl.kernel(
    _msg_body,
    out_type=jax.ShapeDtypeStruct((NC, N_PAD, D_HID), jnp.float32),
    mesh=_MESH,
    compiler_params=_SC_PARAMS,
    scratch_types=[
        pltpu.VMEM((CH, CHUNK), jnp.int32),
        pltpu.VMEM((CH, CHUNK), jnp.int32),
        pltpu.VMEM((CH, CHUNK), jnp.float32),
        pltpu.VMEM((CHUNK, D_HID), jnp.float32),
        pltpu.VMEM((CHUNK, D_HID), jnp.float32),
        pltpu.VMEM((CHUNK, D_HID), jnp.float32),
        pltpu.VMEM_SHARED((N_PAD, D_HID), jnp.float32),
        pltpu.SemaphoreType.DMA,
        pltpu.SemaphoreType.DMA,
        pltpu.SemaphoreType.DMA,
        pltpu.SemaphoreType.DMA,
        pltpu.SemaphoreType.DMA,
        pltpu.SemaphoreType.DMA,
    ],
)


ROWS_BLK = 2000


def _mm1_body(x_ref, w_ref, o_ref):
  o_ref[...] = jnp.dot(x_ref[...], w_ref[...],
                       preferred_element_type=jnp.float32)


def _tc_matmul1(x, w1):
  return pl.pallas_call(
      _mm1_body,
      grid=(N // ROWS_BLK,),
      in_specs=[
          pl.BlockSpec((ROWS_BLK, D_IN), lambda i: (i, 0)),
          pl.BlockSpec((D_IN, D_HID), lambda i: (0, 0)),
      ],
      out_specs=pl.BlockSpec((ROWS_BLK, D_HID), lambda i: (i, 0)),
      out_shape=jax.ShapeDtypeStruct((N, D_HID), jnp.float32),
  )(x, w1)


def _selu_body(acc_ref, b1_ref, o_ref):
  z = acc_ref[0] + acc_ref[1] + b1_ref[...]
  alpha = 1.6732632423543772
  scale = 1.0507009873554805
  o_ref[...] = scale * jnp.where(z > 0, z, alpha * (jnp.exp(z) - 1.0))


def _tc_selu(acc1, b1):
  return pl.pallas_call(
      _selu_body,
      grid=(N // ROWS_BLK,),
      in_specs=[
          pl.BlockSpec((NC, ROWS_BLK, D_HID), lambda i: (0, i, 0)),
          pl.BlockSpec((1, D_HID), lambda i: (0, 0)),
      ],
      out_specs=pl.BlockSpec((ROWS_BLK, D_HID), lambda i: (i, 0)),
      out_shape=jax.ShapeDtypeStruct((N, D_HID), jnp.float32),
  )(acc1, b1)


def _fin_body(acc_ref, w2_ref, b2_ref, o_ref):
  zin = acc_ref[0] + acc_ref[1]
  z = jnp.dot(zin, w2_ref[...], preferred_element_type=jnp.float32)
  z = z + b2_ref[...]
  m = jnp.max(z, axis=-1, keepdims=True)
  ez = jnp.exp(z - m)
  o_ref[...] = ez / jnp.sum(ez, axis=-1, keepdims=True)


def _tc_fin(acc2, w2, b2):
  return pl.pallas_call(
      _fin_body,
      grid=(N // ROWS_BLK,),
      in_specs=[
          pl.BlockSpec((NC, ROWS_BLK, D_HID), lambda i: (0, i, 0)),
          pl.BlockSpec((D_HID, D_OUT), lambda i: (0, 0)),
          pl.BlockSpec((1, D_OUT), lambda i: (0, 0)),
      ],
      out_specs=pl.BlockSpec((ROWS_BLK, D_OUT), lambda i: (i, 0)),
      out_shape=jax.ShapeDtypeStruct((N, D_OUT), jnp.float32),
  )(acc2, w2, b2)


def kernel(x, edge_index, edge_attr, W1, b1, W2, b2):
  pad = E_PAD - E_TOT
  loop = jnp.arange(N, dtype=jnp.int32)
  pad_src = jnp.arange(pad, dtype=jnp.int32) % N
  pad_dst = N + jnp.arange(pad, dtype=jnp.int32) % (N_PAD - N)
  src = jnp.concatenate([edge_index[0], loop, pad_src]).reshape(NW, CH, CHUNK)
  dst = jnp.concatenate([edge_index[1], loop, pad_dst]).reshape(NW, CH, CHUNK)
  ew = jnp.concatenate(
      [edge_attr, jnp.ones((N,), jnp.float32), jnp.zeros((pad,), jnp.float32)]
  ).reshape(NW, CH, CHUNK)

  deg_p = _deg_kernel(dst, ew)
  norm = _norm_kernel(deg_p, src, dst, ew)

  h1 = _tc_matmul1(x, W1)
  acc1 = _msg_kernel(h1, src, dst, norm)
  z = _tc_selu(acc1, b1.reshape(1, D_HID))
  acc2 = _msg_kernel(z, src, dst, norm)
  out = _tc_fin(acc2, W2, b2.reshape(1, D_OUT))
  return out


# speedup vs baseline: 37.6959x; 1.0281x over previous
import jax
import jax.numpy as jnp
from jax import lax
from jax.experimental import pallas as pl
from jax.experimental.pallas import tpu as pltpu
from jax.experimental.pallas import tpu_sc as plsc

N = 10000
E = 320000
D_IN = 128
D_HID = 64
D_OUT = 128

NC = 2
NS = 16
NW = NC * NS
L = 16

N_PAD = 10240
ROWS_PER_TILE = N_PAD // NS
CHUNK = 80
CH = 125
EPW = CH * CHUNK

_MESH = plsc.VectorSubcoreMesh(
    core_axis_name="c", subcore_axis_name="s", num_cores=NC, num_subcores=NS)
_SC_PARAMS = pltpu.CompilerParams(
    needs_layout_passes=False, use_tc_tiling_on_sc=False)

_BCAST_DN = lax.GatherDimensionNumbers(
    offset_dims=(), collapsed_slice_dims=(0,), start_index_map=(0,))


def _lane_bcast(v16, lane):
  idx = jnp.full((L, 1), lane, jnp.int32)
  return lax.gather(v16, idx, _BCAST_DN, slice_sizes=(1,),
                    mode=lax.GatherScatterMode.PROMISE_IN_BOUNDS)


def _wid():
  return lax.axis_index("c") * NS + lax.axis_index("s")


def _deg_body(ei_hbm, ea_hbm, deg_out, dst_v, ew_v, vbuf, deg_sh, sem):
  cid = lax.axis_index("c")
  sid = lax.axis_index("s")
  wid = _wid()
  base = sid * ROWS_PER_TILE

  z = jnp.zeros((L,), jnp.float32)
  def zb(i, _):
    vbuf[pl.ds(i * L, L)] = z
    return 0
  lax.fori_loop(0, ROWS_PER_TILE // L, zb, 0)
  pltpu.sync_copy(vbuf, deg_sh.at[pl.ds(base, ROWS_PER_TILE)])
  plsc.subcore_barrier()

  pltpu.sync_copy(ei_hbm.at[1, wid], dst_v)
  pltpu.sync_copy(ea_hbm.at[wid], ew_v)

  def fire(j, _):
    pltpu.async_copy(ew_v.at[j], deg_sh.at[dst_v.at[j]], sem, add=True)
    return 0
  lax.fori_loop(0, CH, fire, 0)
  def drain(j, _):
    pltpu.make_async_copy(ew_v.at[j], deg_sh.at[dst_v.at[j]], sem).wait()
    return 0
  lax.fori_loop(0, CH, drain, 0)
  plsc.subcore_barrier()

  pltpu.sync_copy(deg_sh.at[pl.ds(base, ROWS_PER_TILE)], vbuf)
  pltpu.sync_copy(vbuf, deg_out.at[cid, pl.ds(base, ROWS_PER_TILE)])


_deg_kernel = pl.kernel(
    _deg_body,
    out_type=jax.ShapeDtypeStruct((NC, N_PAD), jnp.float32),
    mesh=_MESH,
    compiler_params=_SC_PARAMS,
    scratch_types=[
        pltpu.VMEM((CH, CHUNK), jnp.int32),
        pltpu.VMEM((CH, CHUNK), jnp.float32),
        pltpu.VMEM((ROWS_PER_TILE,), jnp.float32),
        pltpu.VMEM_SHARED((N_PAD,), jnp.float32),
        pltpu.SemaphoreType.DMA,
    ],
)


def _rsqrt16(x):
  i = lax.bitcast_convert_type(x, jnp.int32)
  i = jnp.int32(0x5F3759DF) - lax.shift_right_logical(i, 1)
  y = lax.bitcast_convert_type(i, jnp.float32)
  for _ in range(4):
    y = y * (1.5 - 0.5 * x * y * y)
  return y


def _norm_body(deg_hbm, ei_hbm, ea_hbm, norm_out, selfn_out,
               src_v, dst_v, ew_v, norm_v, d0_v, d1_v, dis_v, dis_sh):
  cid = lax.axis_index("c")
  sid = lax.axis_index("s")
  wid = _wid()
  base = sid * ROWS_PER_TILE

  pltpu.sync_copy(deg_hbm.at[0, pl.ds(base, ROWS_PER_TILE)], d0_v)
  pltpu.sync_copy(deg_hbm.at[1, pl.ds(base, ROWS_PER_TILE)], d1_v)

  def disb(i, _):
    sl = pl.ds(i * L, L)
    d = d0_v[sl] + d1_v[sl] + 1.0
    y = _rsqrt16(d)
    d0_v[sl] = y
    d1_v[sl] = y * y
    return 0
  lax.fori_loop(0, ROWS_PER_TILE // L, disb, 0)
  pltpu.sync_copy(d0_v, dis_sh.at[pl.ds(base, ROWS_PER_TILE)])

  @pl.when(cid == 0)
  def _():
    pltpu.sync_copy(d1_v, selfn_out.at[pl.ds(base, ROWS_PER_TILE)])

  plsc.subcore_barrier()

  pltpu.sync_copy(dis_sh, dis_v)
  pltpu.sync_copy(ei_hbm.at[0, wid], src_v)
  pltpu.sync_copy(ei_hbm.at[1, wid], dst_v)
  pltpu.sync_copy(ea_hbm.at[wid], ew_v)

  def chunk(j, _):
    def grp(g, _):
      sl = pl.ds(g * L, L)
      s16 = src_v[j, sl]
      d16 = dst_v[j, sl]
      ds_ = plsc.load_gather(dis_v, [s16])
      dd_ = plsc.load_gather(dis_v, [d16])
      norm_v[j, sl] = ds_ * ew_v[j, sl] * dd_
      return 0
    lax.fori_loop(0, CHUNK // L, grp, 0)
    return 0
  lax.fori_loop(0, CH, chunk, 0)

  pltpu.sync_copy(norm_v, norm_out.at[wid])


_norm_kernel = pl.kernel(
    _norm_body,
    out_type=(jax.ShapeDtypeStruct((NW, CH, CHUNK), jnp.float32),
              jax.ShapeDtypeStruct((N_PAD,), jnp.float32)),
    mesh=_MESH,
    compiler_params=_SC_PARAMS,
    scratch_types=[
        pltpu.VMEM((CH, CHUNK), jnp.int32),
        pltpu.VMEM((CH, CHUNK), jnp.int32),
        pltpu.VMEM((CH, CHUNK), jnp.float32),
        pltpu.VMEM((CH, CHUNK), jnp.float32),
        pltpu.VMEM((ROWS_PER_TILE,), jnp.float32),
        pltpu.VMEM((ROWS_PER_TILE,), jnp.float32),
        pltpu.VMEM((N_PAD,), jnp.float32),
        pltpu.VMEM_SHARED((N_PAD,), jnp.float32),
    ],
)


def _msg_body(h_hbm, ei_hbm, norm_hbm, acc_out,
              src_v, dst_v, norm_v, rows0, rows1, rows2,
              acc_sh, gsem0, gsem1, gsem2, ssem0, ssem1, ssem2):
  cid = lax.axis_index("c")
  sid = lax.axis_index("s")
  wid = _wid()
  base = sid * ROWS_PER_TILE

  bufs = (rows0, rows1, rows2)
  gsems = (gsem0, gsem1, gsem2)
  ssems = (ssem0, ssem1, ssem2)

  z = jnp.zeros((L,), jnp.float32)
  def zb(i, _):
    for k in range(D_HID // L):
      rows0[i, pl.ds(k * L, L)] = z
    return 0
  lax.fori_loop(0, CHUNK, zb, 0)
  for k in range(ROWS_PER_TILE // CHUNK):
    pltpu.sync_copy(rows0, acc_sh.at[pl.ds(base + k * CHUNK, CHUNK), :])
  plsc.subcore_barrier()

  pltpu.sync_copy(ei_hbm.at[0, wid], src_v)
  pltpu.sync_copy(ei_hbm.at[1, wid], dst_v)
  pltpu.sync_copy(norm_hbm.at[wid], norm_v)

  def scale(buf, j):
    nk = D_HID // L
    def grp(g16, _):
      n16 = norm_v[j, pl.ds(g16 * L, L)]
      for e16 in range(0, L, 2):
        ea = g16 * L + e16
        eb = ea + 1
        nba = _lane_bcast(n16, e16)
        nbb = _lane_bcast(n16, e16 + 1)
        va = [buf[ea, pl.ds(k * L, L)] for k in range(nk)]
        vb = [buf[eb, pl.ds(k * L, L)] for k in range(nk)]
        for k in range(nk):
          buf[ea, pl.ds(k * L, L)] = va[k] * nba
        for k in range(nk):
          buf[eb, pl.ds(k * L, L)] = vb[k] * nbb
      return 0
    lax.fori_loop(0, CHUNK // L, grp, 0)

  def gather(j, b):
    pltpu.async_copy(h_hbm.at[src_v.at[j]], bufs[b], gsems[b])

  def wait_gather(j, b):
    pltpu.make_async_copy(h_hbm.at[src_v.at[j]], bufs[b], gsems[b]).wait()

  def scatter(j, b):
    pltpu.async_copy(bufs[b], acc_sh.at[dst_v.at[j]], ssems[b], add=True)

  def wait_scatter(j, b):
    pltpu.make_async_copy(bufs[b], acc_sh.at[dst_v.at[j]], ssems[b]).wait()

  def step(j, b, issue_gather, wait_prev_scatter):
    wait_gather(j, b)
    scale(bufs[b], j)
    scatter(j, b)
    if issue_gather:
      b2 = (b + 2) % 3
      if wait_prev_scatter:
        wait_scatter(j - 1, b2)
      gather(j + 2, b2)

  gather(jnp.int32(0), 0)
  gather(jnp.int32(1), 1)

  step(jnp.int32(0), 0, True, False)
  step(jnp.int32(1), 1, True, True)
  step(jnp.int32(2), 2, True, True)

  def group(g, _):
    j0 = g * 3
    step(j0, 0, True, True)
    step(j0 + 1, 1, True, True)
    step(j0 + 2, 2, True, True)
    return 0
  lax.fori_loop(1, CH // 3, group, 0)

  step(jnp.int32(CH - 2), (CH - 2) % 3, False, False)
  step(jnp.int32(CH - 1), (CH - 1) % 3, False, False)
  for j in (CH - 3, CH - 2, CH - 1):
    wait_scatter(jnp.int32(j), j % 3)

  plsc.subcore_barrier()

  for k in range(ROWS_PER_TILE // CHUNK):
    sl = pl.ds(base + k * CHUNK, CHUNK)
    pltpu.sync_copy(acc_sh.at[sl, :], rows0)
    pltpu.sync_copy(rows0, acc_out.at[cid, sl, :])


_msg_kernel = pl.kernel(
    _msg_body,
    out_type=jax.ShapeDtypeStruct((NC, N_PAD, D_HID), jnp.float32),
    mesh=_MESH,
    compiler_params=_SC_PARAMS,
    scratch_types=[
        pltpu.VMEM((CH, CHUNK), jnp.int32),
        pltpu.VMEM((CH, CHUNK), jnp.int32),
        pltpu.VMEM((CH, CHUNK), jnp.float32),
        pltpu.VMEM((CHUNK, D_HID), jnp.float32),
        pltpu.VMEM((CHUNK, D_HID), jnp.float32),
        pltpu.VMEM((CHUNK, D_HID), jnp.float32),
        pltpu.VMEM_SHARED((N_PAD, D_HID), jnp.float32),
        pltpu.SemaphoreType.DMA,
        pltpu.SemaphoreType.DMA,
        pltpu.SemaphoreType.DMA,
        pltpu.SemaphoreType.DMA,
        pltpu.SemaphoreType.DMA,
        pltpu.SemaphoreType.DMA,
    ],
)


ROWS_BLK = 2000


def _mm1_body(x_ref, w_ref, o_ref):
  o_ref[...] = jnp.dot(x_ref[...], w_ref[...],
                       preferred_element_type=jnp.float32)


def _tc_matmul1(x, w1):
  return pl.pallas_call(
      _mm1_body,
      grid=(N // ROWS_BLK,),
      in_specs=[
          pl.BlockSpec((ROWS_BLK, D_IN), lambda i: (i, 0)),
          pl.BlockSpec((D_IN, D_HID), lambda i: (0, 0)),
      ],
      out_specs=pl.BlockSpec((ROWS_BLK, D_HID), lambda i: (i, 0)),
      out_shape=jax.ShapeDtypeStruct((N, D_HID), jnp.float32),
  )(x, w1)


def _selu_body(acc_ref, h_ref, sn_ref, b1_ref, o_ref):
  z = acc_ref[0] + acc_ref[1] + sn_ref[...] * h_ref[...] + b1_ref[...]
  alpha = 1.6732632423543772
  scale = 1.0507009873554805
  o_ref[...] = scale * jnp.where(z > 0, z, alpha * (jnp.exp(z) - 1.0))


def _tc_selu(acc1, h1, selfn, b1):
  return pl.pallas_call(
      _selu_body,
      grid=(N // ROWS_BLK,),
      in_specs=[
          pl.BlockSpec((NC, ROWS_BLK, D_HID), lambda i: (0, i, 0)),
          pl.BlockSpec((ROWS_BLK, D_HID), lambda i: (i, 0)),
          pl.BlockSpec((ROWS_BLK, 1), lambda i: (i, 0)),
          pl.BlockSpec((1, D_HID), lambda i: (0, 0)),
      ],
      out_specs=pl.BlockSpec((ROWS_BLK, D_HID), lambda i: (i, 0)),
      out_shape=jax.ShapeDtypeStruct((N, D_HID), jnp.float32),
  )(acc1, h1, selfn, b1)


def _fin_body(acc_ref, z_ref, sn_ref, w2_ref, b2_ref, o_ref):
  zin = acc_ref[0] + acc_ref[1] + sn_ref[...] * z_ref[...]
  y = jnp.dot(zin, w2_ref[...], preferred_element_type=jnp.float32)
  y = y + b2_ref[...]
  m = jnp.max(y, axis=-1, keepdims=True)
  ey = jnp.exp(y - m)
  o_ref[...] = ey / jnp.sum(ey, axis=-1, keepdims=True)


def _tc_fin(acc2, z, selfn, w2, b2):
  return pl.pallas_call(
      _fin_body,
      grid=(N // ROWS_BLK,),
      in_specs=[
          pl.BlockSpec((NC, ROWS_BLK, D_HID), lambda i: (0, i, 0)),
          pl.BlockSpec((ROWS_BLK, D_HID), lambda i: (i, 0)),
          pl.BlockSpec((ROWS_BLK, 1), lambda i: (i, 0)),
          pl.BlockSpec((D_HID, D_OUT), lambda i: (0, 0)),
          pl.BlockSpec((1, D_OUT), lambda i: (0, 0)),
      ],
      out_specs=pl.BlockSpec((ROWS_BLK, D_OUT), lambda i: (i, 0)),
      out_shape=jax.ShapeDtypeStruct((N, D_OUT), jnp.float32),
  )(acc2, z, selfn, w2, b2)


def kernel(x, edge_index, edge_attr, W1, b1, W2, b2):
  ei4 = edge_index.reshape(2, NW, CH, CHUNK)
  ea3 = edge_attr.reshape(NW, CH, CHUNK)

  deg_p = _deg_kernel(ei4, ea3)
  norm, selfn = _norm_kernel(deg_p, ei4, ea3)
  selfn_n = selfn[:N].reshape(N, 1)

  h1 = _tc_matmul1(x, W1)
  acc1 = _msg_kernel(h1, ei4, norm)
  z = _tc_selu(acc1, h1, selfn_n, b1.reshape(1, D_HID))
  acc2 = _msg_kernel(z, ei4, norm)
  out = _tc_fin(acc2, z, selfn_n, W2, b2.reshape(1, D_OUT))
  return out


# speedup vs baseline: 39.7635x; 1.0548x over previous
import jax
import jax.numpy as jnp
from jax import lax
from jax.experimental import pallas as pl
from jax.experimental.pallas import tpu as pltpu
from jax.experimental.pallas import tpu_sc as plsc

N = 10000
E = 320000
D_IN = 128
D_HID = 64
D_OUT = 128

NC = 2
NS = 16
NW = NC * NS
L = 16

N_PAD = 10240
ROWS_PER_TILE = N_PAD // NS
CHUNK = 128
NCH = E // CHUNK
BCH = NCH // NW
TAILW = NCH - BCH * NW
GRPS = BCH // 3

_MESH = plsc.VectorSubcoreMesh(
    core_axis_name="c", subcore_axis_name="s", num_cores=NC, num_subcores=NS)
_SC_PARAMS = pltpu.CompilerParams(
    needs_layout_passes=False, use_tc_tiling_on_sc=False)

_BCAST_DN = lax.GatherDimensionNumbers(
    offset_dims=(), collapsed_slice_dims=(0,), start_index_map=(0,))


def _lane_bcast(v16, lane):
  idx = jnp.full((L, 1), lane, jnp.int32)
  return lax.gather(v16, idx, _BCAST_DN, slice_sizes=(1,),
                    mode=lax.GatherScatterMode.PROMISE_IN_BOUNDS)


def _wid():
  return lax.axis_index("c") * NS + lax.axis_index("s")


def _deg_body(ei_hbm, ea_hbm, deg_out, dst_v, ew_v, vbuf, deg_sh, sem):
  cid = lax.axis_index("c")
  sid = lax.axis_index("s")
  wid = _wid()
  base = sid * ROWS_PER_TILE

  z = jnp.zeros((L,), jnp.float32)
  def zb(i, _):
    vbuf[pl.ds(i * L, L)] = z
    return 0
  lax.fori_loop(0, ROWS_PER_TILE // L, zb, 0)
  pltpu.sync_copy(vbuf, deg_sh.at[pl.ds(base, ROWS_PER_TILE)])
  plsc.subcore_barrier()

  pltpu.sync_copy(ei_hbm.at[1, pl.ds(wid * BCH, BCH)], dst_v.at[pl.ds(0, BCH)])
  pltpu.sync_copy(ea_hbm.at[pl.ds(wid * BCH, BCH)], ew_v.at[pl.ds(0, BCH)])

  @pl.when(wid < TAILW)
  def _():
    pltpu.sync_copy(ei_hbm.at[1, NW * BCH + wid], dst_v.at[BCH])
    pltpu.sync_copy(ea_hbm.at[NW * BCH + wid], ew_v.at[BCH])

  def fire(j, _):
    pltpu.async_copy(ew_v.at[j], deg_sh.at[dst_v.at[j]], sem, add=True)
    return 0
  lax.fori_loop(0, BCH, fire, 0)

  @pl.when(wid < TAILW)
  def _():
    pltpu.async_copy(ew_v.at[BCH], deg_sh.at[dst_v.at[BCH]], sem, add=True)

  def drain(j, _):
    pltpu.make_async_copy(ew_v.at[j], deg_sh.at[dst_v.at[j]], sem).wait()
    return 0
  lax.fori_loop(0, BCH, drain, 0)

  @pl.when(wid < TAILW)
  def _():
    pltpu.make_async_copy(ew_v.at[BCH], deg_sh.at[dst_v.at[BCH]], sem).wait()
  plsc.subcore_barrier()

  pltpu.sync_copy(deg_sh.at[pl.ds(base, ROWS_PER_TILE)], vbuf)
  pltpu.sync_copy(vbuf, deg_out.at[cid, pl.ds(base, ROWS_PER_TILE)])


_deg_kernel = pl.kernel(
    _deg_body,
    out_type=jax.ShapeDtypeStruct((NC, N_PAD), jnp.float32),
    mesh=_MESH,
    compiler_params=_SC_PARAMS,
    scratch_types=[
        pltpu.VMEM((BCH + 1, CHUNK), jnp.int32),
        pltpu.VMEM((BCH + 1, CHUNK), jnp.float32),
        pltpu.VMEM((ROWS_PER_TILE,), jnp.float32),
        pltpu.VMEM_SHARED((N_PAD,), jnp.float32),
        pltpu.SemaphoreType.DMA,
    ],
)


def _rsqrt16(x):
  i = lax.bitcast_convert_type(x, jnp.int32)
  i = jnp.int32(0x5F3759DF) - lax.shift_right_logical(i, 1)
  y = lax.bitcast_convert_type(i, jnp.float32)
  for _ in range(4):
    y = y * (1.5 - 0.5 * x * y * y)
  return y


def _norm_body(deg_hbm, ei_hbm, ea_hbm, norm_out, selfn_out,
               src_v, dst_v, ew_v, norm_v, d0_v, d1_v, dis_v, dis_sh):
  cid = lax.axis_index("c")
  sid = lax.axis_index("s")
  wid = _wid()
  base = sid * ROWS_PER_TILE

  pltpu.sync_copy(deg_hbm.at[0, pl.ds(base, ROWS_PER_TILE)], d0_v)
  pltpu.sync_copy(deg_hbm.at[1, pl.ds(base, ROWS_PER_TILE)], d1_v)

  def disb(i, _):
    sl = pl.ds(i * L, L)
    d = d0_v[sl] + d1_v[sl] + 1.0
    y = _rsqrt16(d)
    d0_v[sl] = y
    d1_v[sl] = y * y
    return 0
  lax.fori_loop(0, ROWS_PER_TILE // L, disb, 0)
  pltpu.sync_copy(d0_v, dis_sh.at[pl.ds(base, ROWS_PER_TILE)])

  @pl.when(cid == 0)
  def _():
    pltpu.sync_copy(d1_v, selfn_out.at[pl.ds(base, ROWS_PER_TILE)])

  plsc.subcore_barrier()

  pltpu.sync_copy(dis_sh, dis_v)
  pltpu.sync_copy(ei_hbm.at[0, pl.ds(wid * BCH, BCH)], src_v.at[pl.ds(0, BCH)])
  pltpu.sync_copy(ei_hbm.at[1, pl.ds(wid * BCH, BCH)], dst_v.at[pl.ds(0, BCH)])
  pltpu.sync_copy(ea_hbm.at[pl.ds(wid * BCH, BCH)], ew_v.at[pl.ds(0, BCH)])

  @pl.when(wid < TAILW)
  def _():
    pltpu.sync_copy(ei_hbm.at[0, NW * BCH + wid], src_v.at[BCH])
    pltpu.sync_copy(ei_hbm.at[1, NW * BCH + wid], dst_v.at[BCH])
    pltpu.sync_copy(ea_hbm.at[NW * BCH + wid], ew_v.at[BCH])

  def chunk(j, _):
    def grp(g, _):
      sl = pl.ds(g * L, L)
      s16 = src_v[j, sl]
      d16 = dst_v[j, sl]
      ds_ = plsc.load_gather(dis_v, [s16])
      dd_ = plsc.load_gather(dis_v, [d16])
      norm_v[j, sl] = ds_ * ew_v[j, sl] * dd_
      return 0
    lax.fori_loop(0, CHUNK // L, grp, 0)
    return 0
  lax.fori_loop(0, BCH, chunk, 0)

  @pl.when(wid < TAILW)
  def _():
    chunk(jnp.int32(BCH), 0)

  pltpu.sync_copy(norm_v.at[pl.ds(0, BCH)], norm_out.at[pl.ds(wid * BCH, BCH)])

  @pl.when(wid < TAILW)
  def _():
    pltpu.sync_copy(norm_v.at[BCH], norm_out.at[NW * BCH + wid])


_norm_kernel = pl.kernel(
    _norm_body,
    out_type=(jax.ShapeDtypeStruct((NCH, CHUNK), jnp.float32),
              jax.ShapeDtypeStruct((N_PAD,), jnp.float32)),
    mesh=_MESH,
    compiler_params=_SC_PARAMS,
    scratch_types=[
        pltpu.VMEM((BCH + 1, CHUNK), jnp.int32),
        pltpu.VMEM((BCH + 1, CHUNK), jnp.int32),
        pltpu.VMEM((BCH + 1, CHUNK), jnp.float32),
        pltpu.VMEM((BCH + 1, CHUNK), jnp.float32),
        pltpu.VMEM((ROWS_PER_TILE,), jnp.float32),
        pltpu.VMEM((ROWS_PER_TILE,), jnp.float32),
        pltpu.VMEM((N_PAD,), jnp.float32),
        pltpu.VMEM_SHARED((N_PAD,), jnp.float32),
    ],
)


def _msg_body(h_hbm, ei_hbm, norm_hbm, acc_out,
              src_v, dst_v, norm_v, rows0, rows1, rows2,
              acc_sh, gsem0, gsem1, gsem2, ssem0, ssem1, ssem2):
  cid = lax.axis_index("c")
  sid = lax.axis_index("s")
  wid = _wid()
  base = sid * ROWS_PER_TILE

  bufs = (rows0, rows1, rows2)
  gsems = (gsem0, gsem1, gsem2)
  ssems = (ssem0, ssem1, ssem2)

  z = jnp.zeros((L,), jnp.float32)
  def zb(i, _):
    for k in range(D_HID // L):
      rows0[i, pl.ds(k * L, L)] = z
    return 0
  lax.fori_loop(0, CHUNK, zb, 0)
  for k in range(ROWS_PER_TILE // CHUNK):
    pltpu.sync_copy(rows0, acc_sh.at[pl.ds(base + k * CHUNK, CHUNK), :])
  plsc.subcore_barrier()

  pltpu.sync_copy(ei_hbm.at[0, pl.ds(wid * BCH, BCH)], src_v.at[pl.ds(0, BCH)])
  pltpu.sync_copy(ei_hbm.at[1, pl.ds(wid * BCH, BCH)], dst_v.at[pl.ds(0, BCH)])
  pltpu.sync_copy(norm_hbm.at[pl.ds(wid * BCH, BCH)], norm_v.at[pl.ds(0, BCH)])

  @pl.when(wid < TAILW)
  def _():
    pltpu.sync_copy(ei_hbm.at[0, NW * BCH + wid], src_v.at[BCH])
    pltpu.sync_copy(ei_hbm.at[1, NW * BCH + wid], dst_v.at[BCH])
    pltpu.sync_copy(norm_hbm.at[NW * BCH + wid], norm_v.at[BCH])

  def scale(buf, j):
    nk = D_HID // L
    def grp(g16, _):
      n16 = norm_v[j, pl.ds(g16 * L, L)]
      for e16 in range(0, L, 2):
        ea = g16 * L + e16
        eb = ea + 1
        nba = _lane_bcast(n16, e16)
        nbb = _lane_bcast(n16, e16 + 1)
        va = [buf[ea, pl.ds(k * L, L)] for k in range(nk)]
        vb = [buf[eb, pl.ds(k * L, L)] for k in range(nk)]
        for k in range(nk):
          buf[ea, pl.ds(k * L, L)] = va[k] * nba
        for k in range(nk):
          buf[eb, pl.ds(k * L, L)] = vb[k] * nbb
      return 0
    lax.fori_loop(0, CHUNK // L, grp, 0)

  def gather(j, b):
    pltpu.async_copy(h_hbm.at[src_v.at[j]], bufs[b], gsems[b])

  def wait_gather(j, b):
    pltpu.make_async_copy(h_hbm.at[src_v.at[j]], bufs[b], gsems[b]).wait()

  def scatter(j, b):
    pltpu.async_copy(bufs[b], acc_sh.at[dst_v.at[j]], ssems[b], add=True)

  def wait_scatter(j, b):
    pltpu.make_async_copy(bufs[b], acc_sh.at[dst_v.at[j]], ssems[b]).wait()

  def step(j, b, issue_gather, wait_prev_scatter):
    wait_gather(j, b)
    scale(bufs[b], j)
    scatter(j, b)
    if issue_gather:
      b2 = (b + 2) % 3
      if wait_prev_scatter:
        wait_scatter(j - 1, b2)
      gather(j + 2, b2)

  gather(jnp.int32(0), 0)
  gather(jnp.int32(1), 1)

  step(jnp.int32(0), 0, True, False)
  step(jnp.int32(1), 1, True, True)
  step(jnp.int32(2), 2, True, True)

  def group(g, _):
    j0 = g * 3
    step(j0, 0, True, True)
    step(j0 + 1, 1, True, True)
    step(j0 + 2, 2, True, True)
    return 0
  lax.fori_loop(1, GRPS - 1, group, 0)

  step(jnp.int32(BCH - 3), 0, True, True)
  step(jnp.int32(BCH - 2), 1, False, False)
  step(jnp.int32(BCH - 1), 2, False, False)
  for j in (BCH - 3, BCH - 2, BCH - 1):
    wait_scatter(jnp.int32(j), j % 3)

  @pl.when(wid < TAILW)
  def _():
    jt = jnp.int32(BCH)
    gather(jt, 0)
    wait_gather(jt, 0)
    scale(rows0, jt)
    scatter(jt, 0)
    wait_scatter(jt, 0)

  plsc.subcore_barrier()

  for k in range(ROWS_PER_TILE // CHUNK):
    sl = pl.ds(base + k * CHUNK, CHUNK)
    pltpu.sync_copy(acc_sh.at[sl, :], rows0)
    pltpu.sync_copy(rows0, acc_out.at[cid, sl, :])


_msg_kernel = pl.kernel(
    _msg_body,
    out_type=jax.ShapeDtypeStruct((NC, N_PAD, D_HID), jnp.float32),
    mesh=_MESH,
    compiler_params=_SC_PARAMS,
    scratch_types=[
        pltpu.VMEM((BCH + 1, CHUNK), jnp.int32),
        pltpu.VMEM((BCH + 1, CHUNK), jnp.int32),
        pltpu.VMEM((BCH + 1, CHUNK), jnp.float32),
        pltpu.VMEM((CHUNK, D_HID), jnp.float32),
        pltpu.VMEM((CHUNK, D_HID), jnp.float32),
        pltpu.VMEM((CHUNK, D_HID), jnp.float32),
        pltpu.VMEM_SHARED((N_PAD, D_HID), jnp.float32),
        pltpu.SemaphoreType.DMA,
        pltpu.SemaphoreType.DMA,
        pltpu.SemaphoreType.DMA,
        pltpu.SemaphoreType.DMA,
        pltpu.SemaphoreType.DMA,
        pltpu.SemaphoreType.DMA,
    ],
)


ROWS_BLK = 2000


def _mm1_body(x_ref, w_ref, o_ref):
  o_ref[...] = jnp.dot(x_ref[...], w_ref[...],
                       preferred_element_type=jnp.float32)


def _tc_matmul1(x, w1):
  return pl.pallas_call(
      _mm1_body,
      grid=(N // ROWS_BLK,),
      in_specs=[
          pl.BlockSpec((ROWS_BLK, D_IN), lambda i: (i, 0)),
          pl.BlockSpec((D_IN, D_HID), lambda i: (0, 0)),
      ],
      out_specs=pl.BlockSpec((ROWS_BLK, D_HID), lambda i: (i, 0)),
      out_shape=jax.ShapeDtypeStruct((N, D_HID), jnp.float32),
  )(x, w1)


def _selu_body(acc_ref, h_ref, sn_ref, b1_ref, o_ref):
  z = acc_ref[0] + acc_ref[1] + sn_ref[...] * h_ref[...] + b1_ref[...]
  alpha = 1.6732632423543772
  scale = 1.0507009873554805
  o_ref[...] = scale * jnp.where(z > 0, z, alpha * (jnp.exp(z) - 1.0))


def _tc_selu(acc1, h1, selfn, b1):
  return pl.pallas_call(
      _selu_body,
      grid=(N // ROWS_BLK,),
      in_specs=[
          pl.BlockSpec((NC, ROWS_BLK, D_HID), lambda i: (0, i, 0)),
          pl.BlockSpec((ROWS_BLK, D_HID), lambda i: (i, 0)),
          pl.BlockSpec((ROWS_BLK, 1), lambda i: (i, 0)),
          pl.BlockSpec((1, D_HID), lambda i: (0, 0)),
      ],
      out_specs=pl.BlockSpec((ROWS_BLK, D_HID), lambda i: (i, 0)),
      out_shape=jax.ShapeDtypeStruct((N, D_HID), jnp.float32),
  )(acc1, h1, selfn, b1)


def _fin_body(acc_ref, z_ref, sn_ref, w2_ref, b2_ref, o_ref):
  zin = acc_ref[0] + acc_ref[1] + sn_ref[...] * z_ref[...]
  y = jnp.dot(zin, w2_ref[...], preferred_element_type=jnp.float32)
  y = y + b2_ref[...]
  m = jnp.max(y, axis=-1, keepdims=True)
  ey = jnp.exp(y - m)
  o_ref[...] = ey / jnp.sum(ey, axis=-1, keepdims=True)


def _tc_fin(acc2, z, selfn, w2, b2):
  return pl.pallas_call(
      _fin_body,
      grid=(N // ROWS_BLK,),
      in_specs=[
          pl.BlockSpec((NC, ROWS_BLK, D_HID), lambda i: (0, i, 0)),
          pl.BlockSpec((ROWS_BLK, D_HID), lambda i: (i, 0)),
          pl.BlockSpec((ROWS_BLK, 1), lambda i: (i, 0)),
          pl.BlockSpec((D_HID, D_OUT), lambda i: (0, 0)),
          pl.BlockSpec((1, D_OUT), lambda i: (0, 0)),
      ],
      out_specs=pl.BlockSpec((ROWS_BLK, D_OUT), lambda i: (i, 0)),
      out_shape=jax.ShapeDtypeStruct((N, D_OUT), jnp.float32),
  )(acc2, z, selfn, w2, b2)


def kernel(x, edge_index, edge_attr, W1, b1, W2, b2):
  ei3 = edge_index.reshape(2, NCH, CHUNK)
  ea2 = edge_attr.reshape(NCH, CHUNK)

  deg_p = _deg_kernel(ei3, ea2)
  norm, selfn = _norm_kernel(deg_p, ei3, ea2)
  selfn_n = selfn[:N].reshape(N, 1)

  h1 = _tc_matmul1(x, W1)
  acc1 = _msg_kernel(h1, ei3, norm)
  z = _tc_selu(acc1, h1, selfn_n, b1.reshape(1, D_HID))
  acc2 = _msg_kernel(z, ei3, norm)
  out = _tc_fin(acc2, z, selfn_n, W2, b2.reshape(1, D_OUT))
  return out


# speedup vs baseline: 40.7975x; 1.0260x over previous
import jax
import jax.numpy as jnp
from jax import lax
from jax.experimental import pallas as pl
from jax.experimental.pallas import tpu as pltpu
from jax.experimental.pallas import tpu_sc as plsc

N = 10000
E = 320000
D_IN = 128
D_HID = 64
D_OUT = 128

NC = 2
NS = 16
NW = NC * NS
L = 16

N_PAD = 10240
ROWS_PER_TILE = N_PAD // NS
CHUNK = 128
NCH = E // CHUNK
BCH = NCH // NW
TAILW = NCH - BCH * NW
GRPS = BCH // 3

_MESH = plsc.VectorSubcoreMesh(
    core_axis_name="c", subcore_axis_name="s", num_cores=NC, num_subcores=NS)
_SC_PARAMS = pltpu.CompilerParams(
    needs_layout_passes=False, use_tc_tiling_on_sc=False)

_BCAST_DN = lax.GatherDimensionNumbers(
    offset_dims=(), collapsed_slice_dims=(0,), start_index_map=(0,))


def _lane_bcast(v16, lane):
  idx = jnp.full((L, 1), lane, jnp.int32)
  return lax.gather(v16, idx, _BCAST_DN, slice_sizes=(1,),
                    mode=lax.GatherScatterMode.PROMISE_IN_BOUNDS)


def _wid():
  return lax.axis_index("c") * NS + lax.axis_index("s")


def _deg_body(ei_hbm, ea_hbm, deg_out, dst_v, ew_v, vbuf, deg_sh, sem):
  cid = lax.axis_index("c")
  sid = lax.axis_index("s")
  wid = _wid()
  base = sid * ROWS_PER_TILE

  z = jnp.zeros((L,), jnp.float32)
  def zb(i, _):
    vbuf[pl.ds(i * L, L)] = z
    return 0
  lax.fori_loop(0, ROWS_PER_TILE // L, zb, 0)
  pltpu.sync_copy(vbuf, deg_sh.at[pl.ds(base, ROWS_PER_TILE)])
  plsc.subcore_barrier()

  pltpu.sync_copy(ei_hbm.at[1, pl.ds(wid * BCH, BCH)], dst_v.at[pl.ds(0, BCH)])
  pltpu.sync_copy(ea_hbm.at[pl.ds(wid * BCH, BCH)], ew_v.at[pl.ds(0, BCH)])

  @pl.when(wid < TAILW)
  def _():
    pltpu.sync_copy(ei_hbm.at[1, NW * BCH + wid], dst_v.at[BCH])
    pltpu.sync_copy(ea_hbm.at[NW * BCH + wid], ew_v.at[BCH])

  def fire(j, _):
    pltpu.async_copy(ew_v.at[j], deg_sh.at[dst_v.at[j]], sem, add=True)
    return 0
  lax.fori_loop(0, BCH, fire, 0)

  @pl.when(wid < TAILW)
  def _():
    pltpu.async_copy(ew_v.at[BCH], deg_sh.at[dst_v.at[BCH]], sem, add=True)

  def drain(j, _):
    pltpu.make_async_copy(ew_v.at[j], deg_sh.at[dst_v.at[j]], sem).wait()
    return 0
  lax.fori_loop(0, BCH, drain, 0)

  @pl.when(wid < TAILW)
  def _():
    pltpu.make_async_copy(ew_v.at[BCH], deg_sh.at[dst_v.at[BCH]], sem).wait()
  plsc.subcore_barrier()

  pltpu.sync_copy(deg_sh.at[pl.ds(base, ROWS_PER_TILE)], vbuf)
  pltpu.sync_copy(vbuf, deg_out.at[cid, pl.ds(base, ROWS_PER_TILE)])


_deg_kernel = pl.kernel(
    _deg_body,
    out_type=jax.ShapeDtypeStruct((NC, N_PAD), jnp.float32),
    mesh=_MESH,
    compiler_params=_SC_PARAMS,
    scratch_types=[
        pltpu.VMEM((BCH + 1, CHUNK), jnp.int32),
        pltpu.VMEM((BCH + 1, CHUNK), jnp.float32),
        pltpu.VMEM((ROWS_PER_TILE,), jnp.float32),
        pltpu.VMEM_SHARED((N_PAD,), jnp.float32),
        pltpu.SemaphoreType.DMA,
    ],
)


def _rsqrt16(x):
  i = lax.bitcast_convert_type(x, jnp.int32)
  i = jnp.int32(0x5F3759DF) - lax.shift_right_logical(i, 1)
  y = lax.bitcast_convert_type(i, jnp.float32)
  for _ in range(4):
    y = y * (1.5 - 0.5 * x * y * y)
  return y


def _norm_body(deg_hbm, ei_hbm, ea_hbm, norm_out, selfn_out,
               src_v, dst_v, ew_v, norm_v, d0_v, d1_v, dis_v, dis_sh):
  cid = lax.axis_index("c")
  sid = lax.axis_index("s")
  wid = _wid()
  base = sid * ROWS_PER_TILE

  pltpu.sync_copy(deg_hbm.at[0, pl.ds(base, ROWS_PER_TILE)], d0_v)
  pltpu.sync_copy(deg_hbm.at[1, pl.ds(base, ROWS_PER_TILE)], d1_v)

  def disb(i, _):
    sl = pl.ds(i * L, L)
    d = d0_v[sl] + d1_v[sl] + 1.0
    y = _rsqrt16(d)
    d0_v[sl] = y
    d1_v[sl] = y * y
    return 0
  lax.fori_loop(0, ROWS_PER_TILE // L, disb, 0)
  pltpu.sync_copy(d0_v, dis_sh.at[pl.ds(base, ROWS_PER_TILE)])

  @pl.when(cid == 0)
  def _():
    pltpu.sync_copy(d1_v, selfn_out.at[pl.ds(base, ROWS_PER_TILE)])

  plsc.subcore_barrier()

  pltpu.sync_copy(dis_sh, dis_v)
  pltpu.sync_copy(ei_hbm.at[0, pl.ds(wid * BCH, BCH)], src_v.at[pl.ds(0, BCH)])
  pltpu.sync_copy(ei_hbm.at[1, pl.ds(wid * BCH, BCH)], dst_v.at[pl.ds(0, BCH)])
  pltpu.sync_copy(ea_hbm.at[pl.ds(wid * BCH, BCH)], ew_v.at[pl.ds(0, BCH)])

  @pl.when(wid < TAILW)
  def _():
    pltpu.sync_copy(ei_hbm.at[0, NW * BCH + wid], src_v.at[BCH])
    pltpu.sync_copy(ei_hbm.at[1, NW * BCH + wid], dst_v.at[BCH])
    pltpu.sync_copy(ea_hbm.at[NW * BCH + wid], ew_v.at[BCH])

  def chunk(j, _):
    def grp(g, _):
      sl = pl.ds(g * L, L)
      s16 = src_v[j, sl]
      d16 = dst_v[j, sl]
      ds_ = plsc.load_gather(dis_v, [s16])
      dd_ = plsc.load_gather(dis_v, [d16])
      norm_v[j, sl] = ds_ * ew_v[j, sl] * dd_
      return 0
    lax.fori_loop(0, CHUNK // L, grp, 0)
    return 0
  lax.fori_loop(0, BCH, chunk, 0)

  @pl.when(wid < TAILW)
  def _():
    chunk(jnp.int32(BCH), 0)

  pltpu.sync_copy(norm_v.at[pl.ds(0, BCH)], norm_out.at[pl.ds(wid * BCH, BCH)])

  @pl.when(wid < TAILW)
  def _():
    pltpu.sync_copy(norm_v.at[BCH], norm_out.at[NW * BCH + wid])


_norm_kernel = pl.kernel(
    _norm_body,
    out_type=(jax.ShapeDtypeStruct((NCH, CHUNK), jnp.float32),
              jax.ShapeDtypeStruct((N_PAD,), jnp.float32)),
    mesh=_MESH,
    compiler_params=_SC_PARAMS,
    scratch_types=[
        pltpu.VMEM((BCH + 1, CHUNK), jnp.int32),
        pltpu.VMEM((BCH + 1, CHUNK), jnp.int32),
        pltpu.VMEM((BCH + 1, CHUNK), jnp.float32),
        pltpu.VMEM((BCH + 1, CHUNK), jnp.float32),
        pltpu.VMEM((ROWS_PER_TILE,), jnp.float32),
        pltpu.VMEM((ROWS_PER_TILE,), jnp.float32),
        pltpu.VMEM((N_PAD,), jnp.float32),
        pltpu.VMEM_SHARED((N_PAD,), jnp.float32),
    ],
)


def _msg_body(h_hbm, ei_hbm, norm_hbm, acc_out,
              src_v, dst_v, norm_v, rows0, rows1, rows2,
              acc_sh, gsem0, gsem1, gsem2, ssem0, ssem1, ssem2):
  cid = lax.axis_index("c")
  sid = lax.axis_index("s")
  wid = _wid()
  base = sid * ROWS_PER_TILE

  bufs = (rows0, rows1, rows2)
  gsems = (gsem0, gsem1, gsem2)
  ssems = (ssem0, ssem1, ssem2)

  z = jnp.zeros((L,), jnp.float32)
  def zb(i, _):
    for k in range(D_HID // L):
      rows0[i, pl.ds(k * L, L)] = z
    return 0
  lax.fori_loop(0, CHUNK, zb, 0)
  for k in range(ROWS_PER_TILE // CHUNK):
    pltpu.sync_copy(rows0, acc_sh.at[pl.ds(base + k * CHUNK, CHUNK), :])
  plsc.subcore_barrier()

  pltpu.sync_copy(ei_hbm.at[0, pl.ds(wid * BCH, BCH)], src_v.at[pl.ds(0, BCH)])
  pltpu.sync_copy(ei_hbm.at[1, pl.ds(wid * BCH, BCH)], dst_v.at[pl.ds(0, BCH)])
  pltpu.sync_copy(norm_hbm.at[pl.ds(wid * BCH, BCH)], norm_v.at[pl.ds(0, BCH)])

  @pl.when(wid < TAILW)
  def _():
    pltpu.sync_copy(ei_hbm.at[0, NW * BCH + wid], src_v.at[BCH])
    pltpu.sync_copy(ei_hbm.at[1, NW * BCH + wid], dst_v.at[BCH])
    pltpu.sync_copy(norm_hbm.at[NW * BCH + wid], norm_v.at[BCH])

  def scale(buf, j):
    nk = D_HID // L
    def grp(g16, _):
      n16 = norm_v[j, pl.ds(g16 * L, L)]
      for e16 in range(0, L, 4):
        es = [g16 * L + e16 + t for t in range(4)]
        nbs = [_lane_bcast(n16, e16 + t) for t in range(4)]
        vs = [[buf[e, pl.ds(k * L, L)] for k in range(nk)] for e in es]
        for t, e in enumerate(es):
          for k in range(nk):
            buf[e, pl.ds(k * L, L)] = vs[t][k] * nbs[t]
      return 0
    lax.fori_loop(0, CHUNK // L, grp, 0)

  def gather(j, b):
    pltpu.async_copy(h_hbm.at[src_v.at[j]], bufs[b], gsems[b])

  def wait_gather(j, b):
    pltpu.make_async_copy(h_hbm.at[src_v.at[j]], bufs[b], gsems[b]).wait()

  def scatter(j, b):
    pltpu.async_copy(bufs[b], acc_sh.at[dst_v.at[j]], ssems[b], add=True)

  def wait_scatter(j, b):
    pltpu.make_async_copy(bufs[b], acc_sh.at[dst_v.at[j]], ssems[b]).wait()

  def step(j, b, issue_gather, wait_prev_scatter):
    wait_gather(j, b)
    scale(bufs[b], j)
    scatter(j, b)
    if issue_gather:
      b2 = (b + 2) % 3
      if wait_prev_scatter:
        wait_scatter(j - 1, b2)
      gather(j + 2, b2)

  gather(jnp.int32(0), 0)
  gather(jnp.int32(1), 1)

  step(jnp.int32(0), 0, True, False)
  step(jnp.int32(1), 1, True, True)
  step(jnp.int32(2), 2, True, True)

  def group(g, _):
    j0 = g * 3
    step(j0, 0, True, True)
    step(j0 + 1, 1, True, True)
    step(j0 + 2, 2, True, True)
    return 0
  lax.fori_loop(1, GRPS - 1, group, 0)

  step(jnp.int32(BCH - 3), 0, True, True)
  step(jnp.int32(BCH - 2), 1, False, False)
  step(jnp.int32(BCH - 1), 2, False, False)
  for j in (BCH - 3, BCH - 2, BCH - 1):
    wait_scatter(jnp.int32(j), j % 3)

  @pl.when(wid < TAILW)
  def _():
    jt = jnp.int32(BCH)
    gather(jt, 0)
    wait_gather(jt, 0)
    scale(rows0, jt)
    scatter(jt, 0)
    wait_scatter(jt, 0)

  plsc.subcore_barrier()

  for k in range(ROWS_PER_TILE // CHUNK):
    sl = pl.ds(base + k * CHUNK, CHUNK)
    pltpu.sync_copy(acc_sh.at[sl, :], rows0)
    pltpu.sync_copy(rows0, acc_out.at[cid, sl, :])


_msg_kernel = pl.kernel(
    _msg_body,
    out_type=jax.ShapeDtypeStruct((NC, N_PAD, D_HID), jnp.float32),
    mesh=_MESH,
    compiler_params=_SC_PARAMS,
    scratch_types=[
        pltpu.VMEM((BCH + 1, CHUNK), jnp.int32),
        pltpu.VMEM((BCH + 1, CHUNK), jnp.int32),
        pltpu.VMEM((BCH + 1, CHUNK), jnp.float32),
        pltpu.VMEM((CHUNK, D_HID), jnp.float32),
        pltpu.VMEM((CHUNK, D_HID), jnp.float32),
        pltpu.VMEM((CHUNK, D_HID), jnp.float32),
        pltpu.VMEM_SHARED((N_PAD, D_HID), jnp.float32),
        pltpu.SemaphoreType.DMA,
        pltpu.SemaphoreType.DMA,
        pltpu.SemaphoreType.DMA,
        pltpu.SemaphoreType.DMA,
        pltpu.SemaphoreType.DMA,
        pltpu.SemaphoreType.DMA,
    ],
)


ROWS_BLK = 2000


def _mm1_body(x_ref, w_ref, o_ref):
  o_ref[...] = jnp.dot(x_ref[...], w_ref[...],
                       preferred_element_type=jnp.float32)


def _tc_matmul1(x, w1):
  return pl.pallas_call(
      _mm1_body,
      grid=(N // ROWS_BLK,),
      in_specs=[
          pl.BlockSpec((ROWS_BLK, D_IN), lambda i: (i, 0)),
          pl.BlockSpec((D_IN, D_HID), lambda i: (0, 0)),
      ],
      out_specs=pl.BlockSpec((ROWS_BLK, D_HID), lambda i: (i, 0)),
      out_shape=jax.ShapeDtypeStruct((N, D_HID), jnp.float32),
  )(x, w1)


def _selu_body(acc_ref, h_ref, sn_ref, b1_ref, o_ref):
  z = acc_ref[0] + acc_ref[1] + sn_ref[...] * h_ref[...] + b1_ref[...]
  alpha = 1.6732632423543772
  scale = 1.0507009873554805
  o_ref[...] = scale * jnp.where(z > 0, z, alpha * (jnp.exp(z) - 1.0))


def _tc_selu(acc1, h1, selfn, b1):
  return pl.pallas_call(
      _selu_body,
      grid=(N // ROWS_BLK,),
      in_specs=[
          pl.BlockSpec((NC, ROWS_BLK, D_HID), lambda i: (0, i, 0)),
          pl.BlockSpec((ROWS_BLK, D_HID), lambda i: (i, 0)),
          pl.BlockSpec((ROWS_BLK, 1), lambda i: (i, 0)),
          pl.BlockSpec((1, D_HID), lambda i: (0, 0)),
      ],
      out_specs=pl.BlockSpec((ROWS_BLK, D_HID), lambda i: (i, 0)),
      out_shape=jax.ShapeDtypeStruct((N, D_HID), jnp.float32),
  )(acc1, h1, selfn, b1)


def _fin_body(acc_ref, z_ref, sn_ref, w2_ref, b2_ref, o_ref):
  zin = acc_ref[0] + acc_ref[1] + sn_ref[...] * z_ref[...]
  y = jnp.dot(zin, w2_ref[...], preferred_element_type=jnp.float32)
  y = y + b2_ref[...]
  m = jnp.max(y, axis=-1, keepdims=True)
  ey = jnp.exp(y - m)
  o_ref[...] = ey / jnp.sum(ey, axis=-1, keepdims=True)


def _tc_fin(acc2, z, selfn, w2, b2):
  return pl.pallas_call(
      _fin_body,
      grid=(N // ROWS_BLK,),
      in_specs=[
          pl.BlockSpec((NC, ROWS_BLK, D_HID), lambda i: (0, i, 0)),
          pl.BlockSpec((ROWS_BLK, D_HID), lambda i: (i, 0)),
          pl.BlockSpec((ROWS_BLK, 1), lambda i: (i, 0)),
          pl.BlockSpec((D_HID, D_OUT), lambda i: (0, 0)),
          pl.BlockSpec((1, D_OUT), lambda i: (0, 0)),
      ],
      out_specs=pl.BlockSpec((ROWS_BLK, D_OUT), lambda i: (i, 0)),
      out_shape=jax.ShapeDtypeStruct((N, D_OUT), jnp.float32),
  )(acc2, z, selfn, w2, b2)


def kernel(x, edge_index, edge_attr, W1, b1, W2, b2):
  ei3 = edge_index.reshape(2, NCH, CHUNK)
  ea2 = edge_attr.reshape(NCH, CHUNK)

  deg_p = _deg_kernel(ei3, ea2)
  norm, selfn = _norm_kernel(deg_p, ei3, ea2)
  selfn_n = selfn[:N].reshape(N, 1)

  h1 = _tc_matmul1(x, W1)
  acc1 = _msg_kernel(h1, ei3, norm)
  z = _tc_selu(acc1, h1, selfn_n, b1.reshape(1, D_HID))
  acc2 = _msg_kernel(z, ei3, norm)
  out = _tc_fin(acc2, z, selfn_n, W2, b2.reshape(1, D_OUT))
  return out
